# baseline (speedup 1.0000x reference)
import jax
import jax.numpy as jnp
from jax import lax
from jax.experimental import pallas as pl
from jax.experimental.pallas import tpu as pltpu
from jax.experimental.pallas import tpu_sc as plsc

VOCAB = 100000
DIM = 64
BATCH = 1024

_SC_INFO = plsc.get_sparse_core_info()
_NC = _SC_INFO.num_cores
_NS = _SC_INFO.num_subcores
_NW = _NC * _NS
_BPW = BATCH // _NW

_VT = 2048


def _sc_gather_body(idx_hbm, table_hbm, x_hbm, idx_v, rows_v, sem):
    wid = lax.axis_index("s") * _NC + lax.axis_index("c")
    base = wid * _BPW
    pltpu.sync_copy(idx_hbm.at[pl.ds(base, _BPW)], idx_v)
    pltpu.async_copy(table_hbm.at[idx_v], rows_v, sem).wait()
    pltpu.sync_copy(rows_v, x_hbm.at[pl.ds(base, _BPW)])


def _sc_gather(input_ids, embed_table):
    mesh = plsc.VectorSubcoreMesh(core_axis_name="c", subcore_axis_name="s")
    fn = pl.kernel(
        _sc_gather_body,
        mesh=mesh,
        compiler_params=pltpu.CompilerParams(use_tc_tiling_on_sc=False),
        out_type=jax.ShapeDtypeStruct((BATCH, DIM), jnp.float32),
        scratch_types=[
            pltpu.VMEM((_BPW,), jnp.int32),
            pltpu.VMEM((_BPW, DIM), jnp.float32),
            pltpu.SemaphoreType.DMA,
        ],
    )
    return fn(input_ids, embed_table)


def _tc_matmul_body(x_ref, w_ref, out_ref, state_ref):
    @pl.when(pl.program_id(0) == 0)
    def _():
        state_ref[...] = jnp.sum(
            x_ref[...], axis=1, keepdims=True) * jnp.float32(1.0 / DIM)

    out_ref[...] = lax.dot_general(
        x_ref[...], w_ref[...],
        (((1,), (1,)), ((), ())),
        preferred_element_type=jnp.float32,
    )


def _tc_matmul(x, lm_head_w):
    grid = (pl.cdiv(VOCAB, _VT),)
    return pl.pallas_call(
        _tc_matmul_body,
        grid=grid,
        in_specs=[
            pl.BlockSpec((BATCH, DIM), lambda i: (0, 0)),
            pl.BlockSpec((_VT, DIM), lambda i: (i, 0)),
        ],
        out_specs=[
            pl.BlockSpec((BATCH, _VT), lambda i: (0, i)),
            pl.BlockSpec((BATCH, 1), lambda i: (0, 0)),
        ],
        out_shape=[
            jax.ShapeDtypeStruct((BATCH, VOCAB), jnp.float32),
            jax.ShapeDtypeStruct((BATCH, 1), jnp.float32),
        ],
    )(x, lm_head_w)


def kernel(input_ids, embed_table, lm_head_w):
    ids = input_ids.astype(jnp.int32)
    x = _sc_gather(ids, embed_table)
    logits, state = _tc_matmul(x, lm_head_w)
    return (logits, state.reshape(BATCH))


# speedup vs baseline: 1.0581x; 1.0581x over previous
import jax
import jax.numpy as jnp
from jax import lax
from jax.experimental import pallas as pl
from jax.experimental.pallas import tpu as pltpu
from jax.experimental.pallas import tpu_sc as plsc

VOCAB = 100000
DIM = 64
BATCH = 1024

_SC_INFO = plsc.get_sparse_core_info()
_NC = _SC_INFO.num_cores
_NS = _SC_INFO.num_subcores
_NW = _NC * _NS
_BPW = BATCH // _NW

_VT = 2048


def _sc_gather_body(idx_hbm, table_hbm, x_hbm, idx_v, rows_v, sem):
    wid = lax.axis_index("s") * _NC + lax.axis_index("c")
    base = wid * _BPW
    pltpu.sync_copy(idx_hbm.at[pl.ds(base, _BPW)], idx_v)
    pltpu.async_copy(table_hbm.at[idx_v], rows_v, sem).wait()
    pltpu.sync_copy(rows_v, x_hbm.at[pl.ds(base, _BPW)])


def _sc_gather(input_ids, embed_table):
    mesh = plsc.VectorSubcoreMesh(core_axis_name="c", subcore_axis_name="s")
    fn = pl.kernel(
        _sc_gather_body,
        mesh=mesh,
        compiler_params=pltpu.CompilerParams(use_tc_tiling_on_sc=False),
        out_type=jax.ShapeDtypeStruct((BATCH, DIM), jnp.float32),
        scratch_types=[
            pltpu.VMEM((_BPW,), jnp.int32),
            pltpu.VMEM((_BPW, DIM), jnp.float32),
            pltpu.SemaphoreType.DMA,
        ],
    )
    return fn(input_ids, embed_table)


def _tc_matmul_body(x_ref, w_ref, out_ref, state_ref):
    @pl.when(pl.program_id(0) == 0)
    def _():
        state_ref[...] = jnp.sum(
            x_ref[...], axis=1, keepdims=True) * jnp.float32(1.0 / DIM)

    out_ref[...] = lax.dot_general(
        x_ref[...], w_ref[...],
        (((1,), (1,)), ((), ())),
        preferred_element_type=jnp.float32,
    )


def _tc_matmul(x, lm_head_w):
    grid = (pl.cdiv(VOCAB, _VT),)
    return pl.pallas_call(
        _tc_matmul_body,
        grid=grid,
        in_specs=[
            pl.BlockSpec((BATCH, DIM), lambda i: (0, 0)),
            pl.BlockSpec((_VT, DIM), lambda i: (i, 0)),
        ],
        out_specs=[
            pl.BlockSpec((BATCH, _VT), lambda i: (0, i)),
            pl.BlockSpec((BATCH, 1), lambda i: (0, 0)),
        ],
        out_shape=[
            jax.ShapeDtypeStruct((BATCH, VOCAB), jnp.float32),
            jax.ShapeDtypeStruct((BATCH, 1), jnp.float32),
        ],
    )(x, lm_head_w)


def kernel(input_ids, embed_table, lm_head_w):
    ids = input_ids.astype(jnp.int32)
    x = jnp.take(embed_table, ids, axis=0)
    logits, state = _tc_matmul(x, lm_head_w)
    return (logits, state.reshape(BATCH))
